# baseline (speedup 1.0000x reference)
import functools

import jax
import jax.numpy as jnp
from jax import lax
from jax.experimental import pallas as pl
from jax.experimental.pallas import tpu as pltpu
from jax.experimental.pallas import tpu_sc as plsc

F32 = jnp.float32
NC, NS = 2, 16
NW = NC * NS
CH = 128
CNTW = 16


def _make_sc_agg(n_rows: int, d: int, nchunk: int, with_count: bool):
    assert nchunk % 2 == 0
    rpt = ((n_rows + NS * 8 - 1) // (NS * 8)) * 8
    np_rows = rpt * NS

    mesh = plsc.VectorSubcoreMesh(core_axis_name="c", subcore_axis_name="s",
                                  num_cores=NC, num_subcores=NS)

    out_type = [jax.ShapeDtypeStruct((np_rows, d), F32),
                jax.ShapeDtypeStruct((np_rows, d), F32)]
    scratch = [
        pltpu.VMEM((2, 2, CH), jnp.int32),
        pltpu.VMEM((CH, d), F32),
        pltpu.VMEM((CH, d), F32),
        pltpu.VMEM_SHARED((np_rows, d), F32),
        pltpu.SemaphoreType.DMA,
        pltpu.SemaphoreType.DMA,
        pltpu.SemaphoreType.DMA,
        pltpu.SemaphoreType.DMA,
    ]
    if with_count:
        out_type += [jax.ShapeDtypeStruct((NW, np_rows), F32)]
        scratch += [pltpu.VMEM((np_rows,), F32)]

    def body(*refs):
        if with_count:
            (table, edges, zrow,
             acc0_o, acc1_o, cnt_o,
             idx, rows0, rows1, acc_sh, gsem0, gsem1, ssem0, ssem1,
             cnt_l) = refs
        else:
            (table, edges, zrow,
             acc0_o, acc1_o,
             idx, rows0, rows1, acc_sh, gsem0, gsem1, ssem0, ssem1) = refs
        rows = (rows0, rows1)
        gsem = (gsem0, gsem1)
        ssem = (ssem0, ssem1)
        cid = lax.axis_index("c")
        sid = lax.axis_index("s")
        wid = cid * NS + sid

        pltpu.sync_copy(zrow, acc_sh.at[pl.ds(sid * rpt, rpt)])
        if with_count:
            zv = jnp.zeros((16,), F32)

            def zstep(i, carry):
                cnt_l[pl.ds(i * 16, 16)] = zv
                return carry

            lax.fori_loop(0, np_rows // 16, zstep, 0)
        plsc.subcore_barrier()

        ones16 = jnp.ones((16,), F32)

        def stage(j, b):
            pltpu.sync_copy(edges.at[wid, j], idx.at[b])
            pltpu.async_copy(table.at[idx.at[b, 0]], rows[b], gsem[b])

        def gwait(b):
            pltpu.make_async_copy(table.at[idx.at[b, 0]], rows[b],
                                  gsem[b]).wait()

        def sstart(b):
            pltpu.async_copy(rows[b], acc_sh.at[idx.at[b, 1]], ssem[b],
                             add=True)

        def swait(b):
            pltpu.make_async_copy(rows[b], acc_sh.at[idx.at[b, 1]],
                                  ssem[b]).wait()

        def counts(b):
            if with_count:
                for k in range(CH // 16):
                    dv = idx[b, 1, pl.ds(k * 16, 16)]
                    plsc.addupdate_scatter(cnt_l, [dv], ones16)

        stage(0, 0)
        gwait(0)
        counts(0)
        sstart(0)
        stage(1, 1)

        def pair(g, carry):
            for b in (1, 0):
                j = 2 * g + 2 - b
                nb = 1 - b
                gwait(b)
                counts(b)
                sstart(b)
                swait(nb)
                stage(j + 1, nb)
            return carry

        lax.fori_loop(0, (nchunk - 2) // 2, pair, 0)
        gwait(1)
        counts(1)
        sstart(1)
        swait(0)
        swait(1)
        plsc.subcore_barrier()

        sl = pl.ds(sid * rpt, rpt)
        if with_count:
            pltpu.sync_copy(cnt_l, cnt_o.at[wid])

        @pl.when(cid == 0)
        def _():
            pltpu.sync_copy(acc_sh.at[sl], acc0_o.at[sl])

        @pl.when(cid == 1)
        def _():
            pltpu.sync_copy(acc_sh.at[sl], acc1_o.at[sl])

    return pl.kernel(body, out_type=out_type, mesh=mesh, scratch_types=scratch,
                     compiler_params=pltpu.CompilerParams(
                         use_tc_tiling_on_sc=False, needs_layout_passes=False))


def _tc_mid(x, acc0, acc1, cntT, w1l, b1, w1r, w2l, *, bn):
    n, f = x.shape
    h_dim = w1l.shape[1]
    p_dim = w2l.shape[1]
    grid = (n // bn,)

    def body(x_r, a0_r, a1_r, c_r, w1l_r, b1_r, w1r_r, w2l_r, h_r, p_r):
        cnt = jnp.sum(c_r[...], axis=1, keepdims=True)
        recip = 1.0 / jnp.maximum(cnt, 1.0)
        mean = (a0_r[...] + a1_r[...]) * recip
        h = jnp.dot(mean, w1l_r[...], preferred_element_type=F32) + b1_r[...]
        h = h + jnp.dot(x_r[...], w1r_r[...], preferred_element_type=F32)
        h = jnp.maximum(h, 0.0)
        h_r[...] = h
        p_r[...] = jnp.dot(h, w2l_r[...], preferred_element_type=F32)

    return pl.pallas_call(
        body,
        grid=grid,
        in_specs=[
            pl.BlockSpec((bn, f), lambda i: (i, 0)),
            pl.BlockSpec((bn, f), lambda i: (i, 0)),
            pl.BlockSpec((bn, f), lambda i: (i, 0)),
            pl.BlockSpec((bn, NW), lambda i: (i, 0)),
            pl.BlockSpec((f, h_dim), lambda i: (0, 0)),
            pl.BlockSpec((1, h_dim), lambda i: (0, 0)),
            pl.BlockSpec((f, h_dim), lambda i: (0, 0)),
            pl.BlockSpec((h_dim, p_dim), lambda i: (0, 0)),
        ],
        out_specs=[
            pl.BlockSpec((bn, h_dim), lambda i: (i, 0)),
            pl.BlockSpec((bn, p_dim), lambda i: (i, 0)),
        ],
        out_shape=[jax.ShapeDtypeStruct((n, h_dim), F32),
                   jax.ShapeDtypeStruct((n, p_dim), F32)],
    )(x, acc0, acc1, cntT, w1l, b1, w1r, w2l)


def _tc_out(h, p0, p1, cntT, w2r, b2, *, bn):
    n, h_dim = h.shape
    p_dim = w2r.shape[1]
    grid = (n // bn,)

    def body(h_r, p0_r, p1_r, c_r, w2r_r, b2_r, o_r):
        cnt = jnp.sum(c_r[...], axis=1, keepdims=True)
        recip = 1.0 / jnp.maximum(cnt, 1.0)
        meanp = (p0_r[...] + p1_r[...]) * recip
        o_r[...] = meanp + b2_r[...] + jnp.dot(
            h_r[...], w2r_r[...], preferred_element_type=F32)

    return pl.pallas_call(
        body,
        grid=grid,
        in_specs=[
            pl.BlockSpec((bn, h_dim), lambda i: (i, 0)),
            pl.BlockSpec((bn, p_dim), lambda i: (i, 0)),
            pl.BlockSpec((bn, p_dim), lambda i: (i, 0)),
            pl.BlockSpec((bn, NW), lambda i: (i, 0)),
            pl.BlockSpec((h_dim, p_dim), lambda i: (0, 0)),
            pl.BlockSpec((1, p_dim), lambda i: (0, 0)),
        ],
        out_specs=pl.BlockSpec((bn, p_dim), lambda i: (i, 0)),
        out_shape=jax.ShapeDtypeStruct((n, p_dim), F32),
    )(h, p0, p1, cntT, w2r, b2)


def kernel(x, edge_index, W1l, b1, W1r, W2l, b2, W2r):
    n, f = x.shape
    e = edge_index.shape[1]
    o = W2l.shape[1]
    p_dim = 128
    bn = 400

    nchunk = -(-e // (NW * CH))
    nchunk += nchunk % 2
    e_pad = nchunk * CH * NW
    src = jnp.concatenate(
        [edge_index[0], jnp.zeros((e_pad - e,), jnp.int32)]).reshape(
            NW, nchunk, 1, CH)
    dst = jnp.concatenate(
        [edge_index[1], jnp.full((e_pad - e,), n, jnp.int32)]).reshape(
            NW, nchunk, 1, CH)
    pad_chunk = jnp.concatenate(
        [jnp.zeros((NW, 1, 1, CH), jnp.int32),
         jnp.full((NW, 1, 1, CH), n, jnp.int32)], axis=2)
    edges = jnp.concatenate(
        [jnp.concatenate([src, dst], axis=2), pad_chunk], axis=1)

    rpt = ((n + NS * 8 - 1) // (NS * 8)) * 8
    zrow = jnp.zeros((rpt, f), F32)

    agg1 = _make_sc_agg(n, f, nchunk, with_count=True)
    acc0, acc1, cnt_part = agg1(x, edges, zrow)
    cntT = cnt_part.T

    w2l_p = jnp.pad(W2l, ((0, 0), (0, p_dim - o)))
    h, p = _tc_mid(x, acc0, acc1, cntT,
                   W1l, b1.reshape(1, -1), W1r, w2l_p, bn=bn)

    agg2 = _make_sc_agg(n, p_dim, nchunk, with_count=False)
    pacc0, pacc1 = agg2(p, edges, zrow)

    w2r_p = jnp.pad(W2r, ((0, 0), (0, p_dim - o)))
    b2_p = jnp.pad(b2, (0, p_dim - o)).reshape(1, -1)
    out = _tc_out(h, pacc0, pacc1, cntT, w2r_p, b2_p, bn=bn)
    return out[:, :o]


# speedup vs baseline: 1.7791x; 1.7791x over previous
import functools

import jax
import jax.numpy as jnp
from jax import lax
from jax.experimental import pallas as pl
from jax.experimental.pallas import tpu as pltpu
from jax.experimental.pallas import tpu_sc as plsc

F32 = jnp.float32
NC, NS = 2, 16
NW = NC * NS
CH = 144
CNTW = 16


def _make_sc_agg(n_rows: int, d: int, nchunk: int, with_count: bool):
    assert nchunk % 2 == 0
    rpt = ((n_rows + NS * 8 - 1) // (NS * 8)) * 8
    np_rows = rpt * NS

    mesh = plsc.VectorSubcoreMesh(core_axis_name="c", subcore_axis_name="s",
                                  num_cores=NC, num_subcores=NS)

    out_type = [jax.ShapeDtypeStruct((np_rows, d), F32),
                jax.ShapeDtypeStruct((np_rows, d), F32)]
    scratch = [
        pltpu.VMEM((2, 2, CH), jnp.int32),
        pltpu.VMEM((CH, d), F32),
        pltpu.VMEM((CH, d), F32),
        pltpu.VMEM_SHARED((np_rows, d), F32),
        pltpu.SemaphoreType.DMA,
        pltpu.SemaphoreType.DMA,
        pltpu.SemaphoreType.DMA,
        pltpu.SemaphoreType.DMA,
    ]
    if with_count:
        out_type += [jax.ShapeDtypeStruct((NW, np_rows), F32)]
        scratch += [pltpu.VMEM((np_rows,), F32)]

    def body(*refs):
        if with_count:
            (table, edges, zrow,
             acc0_o, acc1_o, cnt_o,
             idx, rows0, rows1, acc_sh, gsem0, gsem1, ssem0, ssem1,
             cnt_l) = refs
        else:
            (table, edges, zrow,
             acc0_o, acc1_o,
             idx, rows0, rows1, acc_sh, gsem0, gsem1, ssem0, ssem1) = refs
        rows = (rows0, rows1)
        gsem = (gsem0, gsem1)
        ssem = (ssem0, ssem1)
        cid = lax.axis_index("c")
        sid = lax.axis_index("s")
        wid = cid * NS + sid

        pltpu.sync_copy(zrow, acc_sh.at[pl.ds(sid * rpt, rpt)])
        if with_count:
            zv = jnp.zeros((16,), F32)

            def zstep(i, carry):
                cnt_l[pl.ds(i * 16, 16)] = zv
                return carry

            lax.fori_loop(0, np_rows // 16, zstep, 0)
        plsc.subcore_barrier()

        ones16 = jnp.ones((16,), F32)

        def stage(j, b):
            pltpu.sync_copy(edges.at[wid, j], idx.at[b])
            pltpu.async_copy(table.at[idx.at[b, 0]], rows[b], gsem[b])

        def gwait(b):
            pltpu.make_async_copy(table.at[idx.at[b, 0]], rows[b],
                                  gsem[b]).wait()

        def sstart(b):
            pltpu.async_copy(rows[b], acc_sh.at[idx.at[b, 1]], ssem[b],
                             add=True)

        def swait(b):
            pltpu.make_async_copy(rows[b], acc_sh.at[idx.at[b, 1]],
                                  ssem[b]).wait()

        def counts(b):
            if with_count:
                for k in range(CH // 16):
                    dv = idx[b, 1, pl.ds(k * 16, 16)]
                    plsc.addupdate_scatter(cnt_l, [dv], ones16)

        stage(0, 0)
        gwait(0)
        counts(0)
        sstart(0)
        stage(1, 1)

        def pair(g, carry):
            for b in (1, 0):
                j = 2 * g + 2 - b
                nb = 1 - b
                gwait(b)
                counts(b)
                sstart(b)
                swait(nb)
                stage(j + 1, nb)
            return carry

        lax.fori_loop(0, (nchunk - 2) // 2, pair, 0)
        gwait(1)
        counts(1)
        sstart(1)
        swait(0)
        swait(1)
        plsc.subcore_barrier()

        sl = pl.ds(sid * rpt, rpt)
        if with_count:
            pltpu.sync_copy(cnt_l, cnt_o.at[wid])

        @pl.when(cid == 0)
        def _():
            pltpu.sync_copy(acc_sh.at[sl], acc0_o.at[sl])

        @pl.when(cid == 1)
        def _():
            pltpu.sync_copy(acc_sh.at[sl], acc1_o.at[sl])

    return pl.kernel(body, out_type=out_type, mesh=mesh, scratch_types=scratch,
                     compiler_params=pltpu.CompilerParams(
                         use_tc_tiling_on_sc=False, needs_layout_passes=False))


def _tc_mid(x, acc0, acc1, cntT, w1l, b1, w1r, w2l, *, bn):
    n, f = x.shape
    h_dim = w1l.shape[1]
    p_dim = w2l.shape[1]
    grid = (n // bn,)

    def body(x_r, a0_r, a1_r, c_r, w1l_r, b1_r, w1r_r, w2l_r, h_r, p_r):
        cnt = jnp.sum(c_r[...], axis=1, keepdims=True)
        recip = 1.0 / jnp.maximum(cnt, 1.0)
        mean = (a0_r[...] + a1_r[...]) * recip
        h = jnp.dot(mean, w1l_r[...], preferred_element_type=F32) + b1_r[...]
        h = h + jnp.dot(x_r[...], w1r_r[...], preferred_element_type=F32)
        h = jnp.maximum(h, 0.0)
        h_r[...] = h
        p_r[...] = jnp.dot(h, w2l_r[...], preferred_element_type=F32)

    return pl.pallas_call(
        body,
        grid=grid,
        in_specs=[
            pl.BlockSpec((bn, f), lambda i: (i, 0)),
            pl.BlockSpec((bn, f), lambda i: (i, 0)),
            pl.BlockSpec((bn, f), lambda i: (i, 0)),
            pl.BlockSpec((bn, NW), lambda i: (i, 0)),
            pl.BlockSpec((f, h_dim), lambda i: (0, 0)),
            pl.BlockSpec((1, h_dim), lambda i: (0, 0)),
            pl.BlockSpec((f, h_dim), lambda i: (0, 0)),
            pl.BlockSpec((h_dim, p_dim), lambda i: (0, 0)),
        ],
        out_specs=[
            pl.BlockSpec((bn, h_dim), lambda i: (i, 0)),
            pl.BlockSpec((bn, p_dim), lambda i: (i, 0)),
        ],
        out_shape=[jax.ShapeDtypeStruct((n, h_dim), F32),
                   jax.ShapeDtypeStruct((n, p_dim), F32)],
    )(x, acc0, acc1, cntT, w1l, b1, w1r, w2l)


def _tc_out(h, p0, p1, cntT, w2r, b2, *, bn):
    n, h_dim = h.shape
    p_dim = w2r.shape[1]
    grid = (n // bn,)

    def body(h_r, p0_r, p1_r, c_r, w2r_r, b2_r, o_r):
        cnt = jnp.sum(c_r[...], axis=1, keepdims=True)
        recip = 1.0 / jnp.maximum(cnt, 1.0)
        meanp = (p0_r[...] + p1_r[...]) * recip
        o_r[...] = meanp + b2_r[...] + jnp.dot(
            h_r[...], w2r_r[...], preferred_element_type=F32)

    return pl.pallas_call(
        body,
        grid=grid,
        in_specs=[
            pl.BlockSpec((bn, h_dim), lambda i: (i, 0)),
            pl.BlockSpec((bn, p_dim), lambda i: (i, 0)),
            pl.BlockSpec((bn, p_dim), lambda i: (i, 0)),
            pl.BlockSpec((bn, NW), lambda i: (i, 0)),
            pl.BlockSpec((h_dim, p_dim), lambda i: (0, 0)),
            pl.BlockSpec((1, p_dim), lambda i: (0, 0)),
        ],
        out_specs=pl.BlockSpec((bn, p_dim), lambda i: (i, 0)),
        out_shape=jax.ShapeDtypeStruct((n, p_dim), F32),
    )(h, p0, p1, cntT, w2r, b2)


def kernel(x, edge_index, W1l, b1, W1r, W2l, b2, W2r):
    n, f = x.shape
    e = edge_index.shape[1]
    o = W2l.shape[1]
    p_dim = 128
    bn = 1000

    nchunk = -(-e // (NW * CH))
    nchunk += nchunk % 2
    e_pad = nchunk * CH * NW
    src = jnp.concatenate(
        [edge_index[0], jnp.zeros((e_pad - e,), jnp.int32)]).reshape(
            NW, nchunk, 1, CH)
    dst = jnp.concatenate(
        [edge_index[1], jnp.full((e_pad - e,), n, jnp.int32)]).reshape(
            NW, nchunk, 1, CH)
    pad_chunk = jnp.concatenate(
        [jnp.zeros((NW, 1, 1, CH), jnp.int32),
         jnp.full((NW, 1, 1, CH), n, jnp.int32)], axis=2)
    edges = jnp.concatenate(
        [jnp.concatenate([src, dst], axis=2), pad_chunk], axis=1)

    rpt = ((n + NS * 8 - 1) // (NS * 8)) * 8
    zrow = jnp.zeros((rpt, f), F32)

    agg1 = _make_sc_agg(n, f, nchunk, with_count=True)
    acc0, acc1, cnt_part = agg1(x, edges, zrow)
    cntT = cnt_part.T

    w2l_p = jnp.pad(W2l, ((0, 0), (0, p_dim - o)))
    h, p = _tc_mid(x, acc0, acc1, cntT,
                   W1l, b1.reshape(1, -1), W1r, w2l_p, bn=bn)

    agg2 = _make_sc_agg(n, p_dim, nchunk, with_count=False)
    pacc0, pacc1 = agg2(p, edges, zrow)

    w2r_p = jnp.pad(W2r, ((0, 0), (0, p_dim - o)))
    b2_p = jnp.pad(b2, (0, p_dim - o)).reshape(1, -1)
    out = _tc_out(h, pacc0, pacc1, cntT, w2r_p, b2_p, bn=bn)
    return out[:, :o]


# speedup vs baseline: 1.7904x; 1.0064x over previous
import functools

import jax
import jax.numpy as jnp
from jax import lax
from jax.experimental import pallas as pl
from jax.experimental.pallas import tpu as pltpu
from jax.experimental.pallas import tpu_sc as plsc

F32 = jnp.float32
NC, NS = 2, 16
NW = NC * NS
CH = 144
CNTW = 16


def _make_sc_agg(n_rows: int, d: int, nchunk: int, with_count: bool):
    assert nchunk % 2 == 0
    rpt = ((n_rows + NS * 8 - 1) // (NS * 8)) * 8
    np_rows = rpt * NS

    mesh = plsc.VectorSubcoreMesh(core_axis_name="c", subcore_axis_name="s",
                                  num_cores=NC, num_subcores=NS)

    out_type = [jax.ShapeDtypeStruct((np_rows, d), F32),
                jax.ShapeDtypeStruct((np_rows, d), F32)]
    scratch = [
        pltpu.VMEM((2, 2, CH), jnp.int32),
        pltpu.VMEM((CH, d), F32),
        pltpu.VMEM((CH, d), F32),
        pltpu.VMEM_SHARED((np_rows, d), F32),
        pltpu.SemaphoreType.DMA,
        pltpu.SemaphoreType.DMA,
        pltpu.SemaphoreType.DMA,
        pltpu.SemaphoreType.DMA,
    ]
    if with_count:
        out_type += [jax.ShapeDtypeStruct((NW, np_rows), F32)]
        scratch += [pltpu.VMEM((np_rows,), F32)]

    def body(*refs):
        if with_count:
            (table, edges, zrow,
             acc0_o, acc1_o, cnt_o,
             idx, rows0, rows1, acc_sh, gsem0, gsem1, ssem0, ssem1,
             cnt_l) = refs
        else:
            (table, edges, zrow,
             acc0_o, acc1_o,
             idx, rows0, rows1, acc_sh, gsem0, gsem1, ssem0, ssem1) = refs
        rows = (rows0, rows1)
        gsem = (gsem0, gsem1)
        ssem = (ssem0, ssem1)
        cid = lax.axis_index("c")
        sid = lax.axis_index("s")
        wid = cid * NS + sid

        pltpu.sync_copy(zrow, acc_sh.at[pl.ds(sid * rpt, rpt)])
        if with_count:
            zv = jnp.zeros((16,), F32)

            def zstep(i, carry):
                cnt_l[pl.ds(i * 16, 16)] = zv
                return carry

            lax.fori_loop(0, np_rows // 16, zstep, 0)
        plsc.subcore_barrier()

        ones16 = jnp.ones((16,), F32)

        def stage(j, b):
            pltpu.sync_copy(edges.at[wid, j], idx.at[b])
            pltpu.async_copy(table.at[idx.at[b, 0]], rows[b], gsem[b])

        def gwait(b):
            pltpu.make_async_copy(table.at[idx.at[b, 0]], rows[b],
                                  gsem[b]).wait()

        def sstart(b):
            pltpu.async_copy(rows[b], acc_sh.at[idx.at[b, 1]], ssem[b],
                             add=True)

        def swait(b):
            pltpu.make_async_copy(rows[b], acc_sh.at[idx.at[b, 1]],
                                  ssem[b]).wait()

        def counts(b):
            if with_count:
                for k in range(CH // 16):
                    dv = idx[b, 1, pl.ds(k * 16, 16)]
                    plsc.addupdate_scatter(cnt_l, [dv], ones16)

        stage(0, 0)
        gwait(0)
        counts(0)
        sstart(0)
        stage(1, 1)

        def pair(g, carry):
            for b in (1, 0):
                j = 2 * g + 2 - b
                nb = 1 - b
                gwait(b)
                counts(b)
                sstart(b)
                swait(nb)
                stage(j + 1, nb)
            return carry

        lax.fori_loop(0, (nchunk - 2) // 2, pair, 0)
        gwait(1)
        counts(1)
        sstart(1)
        swait(0)
        swait(1)
        plsc.subcore_barrier()

        sl = pl.ds(sid * rpt, rpt)
        if with_count:
            pltpu.sync_copy(cnt_l, cnt_o.at[wid])

        @pl.when(cid == 0)
        def _():
            pltpu.sync_copy(acc_sh.at[sl], acc0_o.at[sl])

        @pl.when(cid == 1)
        def _():
            pltpu.sync_copy(acc_sh.at[sl], acc1_o.at[sl])

    return pl.kernel(body, out_type=out_type, mesh=mesh, scratch_types=scratch,
                     compiler_params=pltpu.CompilerParams(
                         use_tc_tiling_on_sc=False, needs_layout_passes=False))


def _tc_mid(x, acc0, acc1, cntT, w1l, b1, w1r, w2l, *, bn):
    n, f = x.shape
    h_dim = w1l.shape[1]
    p_dim = w2l.shape[1]
    grid = (n // bn,)

    def body(x_r, a0_r, a1_r, c_r, w1l_r, b1_r, w1r_r, w2l_r, h_r, p_r):
        cnt = jnp.sum(c_r[...], axis=1, keepdims=True)
        recip = 1.0 / jnp.maximum(cnt, 1.0)
        mean = (a0_r[...] + a1_r[...]) * recip
        h = jnp.dot(mean, w1l_r[...], preferred_element_type=F32) + b1_r[...]
        h = h + jnp.dot(x_r[...], w1r_r[...], preferred_element_type=F32)
        h = jnp.maximum(h, 0.0)
        h_r[...] = h
        p_r[...] = jnp.dot(h, w2l_r[...], preferred_element_type=F32)

    return pl.pallas_call(
        body,
        grid=grid,
        in_specs=[
            pl.BlockSpec((bn, f), lambda i: (i, 0)),
            pl.BlockSpec((bn, f), lambda i: (i, 0)),
            pl.BlockSpec((bn, f), lambda i: (i, 0)),
            pl.BlockSpec((bn, NW), lambda i: (i, 0)),
            pl.BlockSpec((f, h_dim), lambda i: (0, 0)),
            pl.BlockSpec((1, h_dim), lambda i: (0, 0)),
            pl.BlockSpec((f, h_dim), lambda i: (0, 0)),
            pl.BlockSpec((h_dim, p_dim), lambda i: (0, 0)),
        ],
        out_specs=[
            pl.BlockSpec((bn, h_dim), lambda i: (i, 0)),
            pl.BlockSpec((bn, p_dim), lambda i: (i, 0)),
        ],
        out_shape=[jax.ShapeDtypeStruct((n, h_dim), F32),
                   jax.ShapeDtypeStruct((n, p_dim), F32)],
    )(x, acc0, acc1, cntT, w1l, b1, w1r, w2l)


def _tc_out(h, p0, p1, cntT, w2r, b2, *, bn):
    n, h_dim = h.shape
    p_dim = w2r.shape[1]
    grid = (n // bn,)

    def body(h_r, p0_r, p1_r, c_r, w2r_r, b2_r, o_r):
        cnt = jnp.sum(c_r[...], axis=1, keepdims=True)
        recip = 1.0 / jnp.maximum(cnt, 1.0)
        meanp = (p0_r[...] + p1_r[...]) * recip
        o_r[...] = meanp + b2_r[...] + jnp.dot(
            h_r[...], w2r_r[...], preferred_element_type=F32)

    return pl.pallas_call(
        body,
        grid=grid,
        in_specs=[
            pl.BlockSpec((bn, h_dim), lambda i: (i, 0)),
            pl.BlockSpec((bn, p_dim), lambda i: (i, 0)),
            pl.BlockSpec((bn, p_dim), lambda i: (i, 0)),
            pl.BlockSpec((bn, NW), lambda i: (i, 0)),
            pl.BlockSpec((h_dim, p_dim), lambda i: (0, 0)),
            pl.BlockSpec((1, p_dim), lambda i: (0, 0)),
        ],
        out_specs=pl.BlockSpec((bn, p_dim), lambda i: (i, 0)),
        out_shape=jax.ShapeDtypeStruct((n, p_dim), F32),
    )(h, p0, p1, cntT, w2r, b2)


def kernel(x, edge_index, W1l, b1, W1r, W2l, b2, W2r):
    n, f = x.shape
    e = edge_index.shape[1]
    o = W2l.shape[1]
    p_dim = 128
    bn = 2000

    nchunk = -(-e // (NW * CH))
    nchunk += nchunk % 2
    e_pad = nchunk * CH * NW
    src = jnp.concatenate(
        [edge_index[0], jnp.zeros((e_pad - e,), jnp.int32)]).reshape(
            NW, nchunk, 1, CH)
    dst = jnp.concatenate(
        [edge_index[1], jnp.full((e_pad - e,), n, jnp.int32)]).reshape(
            NW, nchunk, 1, CH)
    pad_chunk = jnp.concatenate(
        [jnp.zeros((NW, 1, 1, CH), jnp.int32),
         jnp.full((NW, 1, 1, CH), n, jnp.int32)], axis=2)
    edges = jnp.concatenate(
        [jnp.concatenate([src, dst], axis=2), pad_chunk], axis=1)

    rpt = ((n + NS * 8 - 1) // (NS * 8)) * 8
    zrow = jnp.zeros((rpt, f), F32)

    agg1 = _make_sc_agg(n, f, nchunk, with_count=True)
    acc0, acc1, cnt_part = agg1(x, edges, zrow)
    cntT = cnt_part.T

    w2l_p = jnp.pad(W2l, ((0, 0), (0, p_dim - o)))
    h, p = _tc_mid(x, acc0, acc1, cntT,
                   W1l, b1.reshape(1, -1), W1r, w2l_p, bn=bn)

    agg2 = _make_sc_agg(n, p_dim, nchunk, with_count=False)
    pacc0, pacc1 = agg2(p, edges, zrow)

    w2r_p = jnp.pad(W2r, ((0, 0), (0, p_dim - o)))
    b2_p = jnp.pad(b2, (0, p_dim - o)).reshape(1, -1)
    out = _tc_out(h, pacc0, pacc1, cntT, w2r_p, b2_p, bn=bn)
    return out[:, :o]
